# baseline (speedup 1.0000x reference)
import functools

import jax
import jax.numpy as jnp
from jax import lax
from jax.experimental import pallas as pl
from jax.experimental.pallas import tpu as pltpu
from jax.experimental.pallas import tpu_sc as plsc

N = 10000
E = 320000
F = 128
G = 64
ITERS = 3

NC, NS = 2, 16
NW = NC * NS
EPT = E // NW
CH = 80
NCHUNK = EPT // CH
DEGW = 128

RQ = 624
RC = 16
RL = N - (NS - 1) * RQ

@functools.cache
def _mesh():
    return plsc.VectorSubcoreMesh(core_axis_name="c", subcore_axis_name="s",
                                  num_cores=NC, num_subcores=NS)


def _zero_fill(ref, nrows, width):
    z16 = jnp.zeros((16,), jnp.float32)

    def body(i, _):
        for j in range(width // 16):
            ref[i, pl.ds(16 * j, 16)] = z16
        return 0

    lax.fori_loop(0, nrows, body, 0)


def _tile_rows(s):
    start = s * RQ
    nch = jnp.where(s == NS - 1, (N - (NS - 1) * RQ) // RC, RQ // RC)
    return start, nch


H = N // 2


@functools.cache
def _sc_degree():
    return pl.kernel(
        _sc_degree_body,
        out_type=jax.ShapeDtypeStruct((2 * NW, H * 16), jnp.float32),
        mesh=_mesh(),
        scratch_types=[
            pltpu.VMEM((EPT,), jnp.int32),
            pltpu.VMEM((H * 16,), jnp.float32),
        ],
        compiler_params=pltpu.CompilerParams(needs_layout_passes=False),
    )


def _sc_degree_body(dst_hbm, z16_hbm, out_hbm, idx, hist):
    c = lax.axis_index("c")
    s = lax.axis_index("s")
    wid = c * NS + s
    pltpu.sync_copy(dst_hbm.at[pl.ds(wid * EPT, EPT)], idx)
    lanes = lax.iota(jnp.int32, 16)
    ones16 = jnp.ones((16,), jnp.float32)
    for p in range(2):
        lo = p * H
        pltpu.sync_copy(z16_hbm.at[p], hist)

        def body(i, _):
            v = idx[pl.ds(i * 16, 16)]
            m = (v >= lo) & (v < lo + H)
            a = jnp.where(m, v - lo, 0) * 16 + lanes
            plsc.addupdate_scatter(hist, [a], ones16, mask=m)
            return 0

        lax.fori_loop(0, EPT // 16, body, 0)
        pltpu.sync_copy(hist, out_hbm.at[p * NW + wid])


@functools.cache
def _sc_agg():
    return pl.kernel(
        _sc_agg_body,
        out_type=jax.ShapeDtypeStruct((NC * N, F), jnp.float32),
        mesh=_mesh(),
        scratch_types=[
            pltpu.VMEM((NCHUNK, CH), jnp.int32),
            [pltpu.VMEM((CH,), jnp.int32) for _ in range(3)],
            [pltpu.VMEM((CH, F), jnp.float32) for _ in range(3)],
            pltpu.VMEM_SHARED((N, F), jnp.float32),
            [pltpu.SemaphoreType.DMA for _ in range(3)],
            [pltpu.SemaphoreType.DMA for _ in range(3)],
            [pltpu.SemaphoreType.DMA for _ in range(3)],
        ],
    )


def _sc_agg_body(u_hbm, src_hbm, dst_hbm, zeros_hbm, out_hbm, sidx, didx,
                 rows, aggsh, semg, semd, sems):
    c = lax.axis_index("c")
    s = lax.axis_index("s")
    wid = c * NS + s
    base = wid * EPT

    start, _ = _tile_rows(s)

    @pl.when(s < NS - 1)
    def _():
        pltpu.sync_copy(zeros_hbm.at[pl.ds(start, RQ)], aggsh.at[pl.ds(start, RQ)])

    @pl.when(s == NS - 1)
    def _():
        pltpu.sync_copy(zeros_hbm.at[pl.ds(start, RL)], aggsh.at[pl.ds(start, RL)])

    pltpu.sync_copy(src_hbm.at[wid], sidx)
    plsc.subcore_barrier()

    def fire(j, b):
        pltpu.async_copy(u_hbm.at[sidx.at[j]], rows[b], semg[b])
        pltpu.async_copy(dst_hbm.at[pl.ds(base + j * CH, CH)], didx[b], semd[b])

    def gwait(b):
        pltpu.make_async_copy(u_hbm.at[pl.ds(0, CH)], rows[b], semg[b]).wait()
        pltpu.make_async_copy(dst_hbm.at[pl.ds(0, CH)], didx[b], semd[b]).wait()

    def scat(b):
        pltpu.async_copy(rows[b], aggsh.at[didx[b]], sems[b], add=True)

    def swait(b):
        pltpu.make_async_copy(rows[b], aggsh.at[didx[b]], sems[b]).wait()

    fire(0, 0)
    fire(1, 1)

    def triple(k, _):
        for t in range(3):
            j = 3 * k + t
            b = t
            bp = (t + 2) % 3
            gwait(b)
            if t == 0:
                @pl.when(k > 0)
                def _():
                    swait(bp)
            else:
                swait(bp)
            fire(j + 2, bp)
            scat(b)
        return 0

    lax.fori_loop(0, (NCHUNK - 2) // 3, triple, 0)
    gwait(0)
    swait(2)
    scat(0)
    gwait(1)
    swait(0)
    scat(1)
    swait(1)
    plsc.subcore_barrier()

    @pl.when(s < NS - 1)
    def _():
        pltpu.sync_copy(aggsh.at[pl.ds(start, RQ)],
                        out_hbm.at[pl.ds(c * N + start, RQ)])

    @pl.when(s == NS - 1)
    def _():
        pltpu.sync_copy(aggsh.at[pl.ds(start, RL)],
                        out_hbm.at[pl.ds(c * N + start, RL)])


R = 200
NB = N // R

_full = lambda shape: pl.BlockSpec(shape, lambda i: (0, 0))
_rows = lambda w: pl.BlockSpec((R, w), lambda i: (i, 0))
_rows0 = pl.BlockSpec((R, F), lambda i: (i, 0))
_rows1 = pl.BlockSpec((R, F), lambda i: (i + NB, 0))


def _k0a_body(x_ref, d_ref, wet_ref, be_ref, wct_ref,
              y_ref, u_ref, dinv_ref):
    y = jnp.dot(x_ref[...], wet_ref[...], preferred_element_type=jnp.float32)
    y = y + be_ref[...]
    deg = jnp.sum(d_ref[...], axis=(0, 2)).reshape(R, 1) + 1.0
    dinv = lax.rsqrt(deg)
    yw = jnp.dot(y, wct_ref[...], preferred_element_type=jnp.float32)
    y_ref[...] = y
    u_ref[...] = dinv * yw
    dinv_ref[...] = dinv


_BPH = H // R


_k0a = pl.pallas_call(
    _k0a_body,
    grid=(NB,),
    in_specs=[
        _rows0,
        pl.BlockSpec((NW, R, 16), lambda i: (i // _BPH, i % _BPH, 0)),
        _full((F, F)), _full((1, F)), _full((F, F)),
    ],
    out_specs=[_rows(F), _rows(F), _rows(1)],
    out_shape=[
        jax.ShapeDtypeStruct((N, F), jnp.float32),
        jax.ShapeDtypeStruct((N, F), jnp.float32),
        jax.ShapeDtypeStruct((N, 1), jnp.float32),
    ],
)


def _kc_body(y_ref, dinv_ref, wct_ref, wrt_ref, brc_ref, c_ref):
    y = y_ref[...]
    dinv = dinv_ref[...]
    yw = jnp.dot(y, wct_ref[...], preferred_element_type=jnp.float32)
    yr = jnp.dot(y, wrt_ref[...], preferred_element_type=jnp.float32)
    c_ref[...] = (dinv * dinv) * yw - yw + yr + brc_ref[...]


_kc = pl.pallas_call(
    _kc_body,
    grid=(NB,),
    in_specs=[_rows0, _rows(1), _full((F, F)), _full((F, F)), _full((1, F))],
    out_specs=_rows(F),
    out_shape=jax.ShapeDtypeStruct((N, F), jnp.float32),
)


def _kmid_body(h0_ref, h1_ref, cprev_ref, dinv_ref, wct_ref, y_ref, u_ref):
    dinv = dinv_ref[...]
    y = jnp.maximum(dinv * (h0_ref[...] + h1_ref[...]) + cprev_ref[...], 0.0)
    yw = jnp.dot(y, wct_ref[...], preferred_element_type=jnp.float32)
    y_ref[...] = y
    u_ref[...] = dinv * yw


_kmid = pl.pallas_call(
    _kmid_body,
    grid=(NB,),
    in_specs=[_rows0, _rows1, _rows(F), _rows(1), _full((F, F))],
    out_specs=[_rows(F), _rows(F)],
    out_shape=[
        jax.ShapeDtypeStruct((N, F), jnp.float32),
        jax.ShapeDtypeStruct((N, F), jnp.float32),
    ],
)


def _k3_body(h0_ref, h1_ref, c_ref, dinv_ref, b_ref, wo_ref, bo_ref, out_ref):
    y = jnp.maximum(dinv_ref[...] * (h0_ref[...] + h1_ref[...]) + c_ref[...], 0.0)
    v = jnp.dot(y, wo_ref[...], preferred_element_type=jnp.float32) + bo_ref[...]
    gi = lax.broadcasted_iota(jnp.int32, (R, G), 1)
    oh = (b_ref[...] == gi).astype(jnp.float32)
    part = jnp.sum(oh * v, axis=0, keepdims=True)

    @pl.when(pl.program_id(0) == 0)
    def _():
        out_ref[...] = jnp.zeros_like(out_ref)

    out_ref[...] += part


_k3 = pl.pallas_call(
    _k3_body,
    grid=(NB,),
    in_specs=[_rows0, _rows1, _rows(F), _rows(1), _rows(1),
              _full((F, 1)), _full((1, 1))],
    out_specs=pl.BlockSpec((1, G), lambda i: (0, 0)),
    out_shape=jax.ShapeDtypeStruct((1, G), jnp.float32),
)


def kernel(x, edge_index, batch, W_emb, b_emb, W_conv, b_conv, W_res, b_res,
           W_out, b_out):
    src = edge_index[0].astype(jnp.int32)
    dst = edge_index[1].astype(jnp.int32)
    src2 = src.reshape(NW, NCHUNK, CH)
    batch2 = batch.astype(jnp.int32).reshape(N, 1)
    x = x.astype(jnp.float32)
    wet = W_emb.T
    wct = W_conv.T
    wrt = W_res.T
    be = b_emb.reshape(1, F)
    brc = (b_res + b_conv).reshape(1, F)
    wo = W_out.T.reshape(F, 1)
    bo = b_out.reshape(1, 1)

    zrows = jnp.zeros((N, F), jnp.float32)
    z16 = jnp.zeros((2, H * 16), jnp.float32)
    degh = _sc_degree()(dst, z16).reshape(2 * NW, H, 16)
    y, u, dinv = _k0a(x, degh, wet, be, wct)
    for it in range(ITERS):
        h = _sc_agg()(u, src2, dst, zrows)
        c = _kc(y, dinv, wct, wrt, brc)
        if it < ITERS - 1:
            y, u = _kmid(h, h, c, dinv, wct)
        else:
            out = _k3(h, h, c, dinv, batch2, wo, bo)
    return out.reshape(G)


# speedup vs baseline: 1.3570x; 1.3570x over previous
import functools

import jax
import jax.numpy as jnp
from jax import lax
from jax.experimental import pallas as pl
from jax.experimental.pallas import tpu as pltpu
from jax.experimental.pallas import tpu_sc as plsc

N = 10000
E = 320000
F = 128
G = 64
ITERS = 3

NC, NS = 2, 16
NW = NC * NS
EPT = E // NW
CH = 80
NCHUNK = EPT // CH
DEGW = 128

RQ = 624
RC = 16
RL = N - (NS - 1) * RQ

@functools.cache
def _mesh():
    return plsc.VectorSubcoreMesh(core_axis_name="c", subcore_axis_name="s",
                                  num_cores=NC, num_subcores=NS)


def _zero_fill(ref, nrows, width):
    z16 = jnp.zeros((16,), jnp.float32)

    def body(i, _):
        for j in range(width // 16):
            ref[i, pl.ds(16 * j, 16)] = z16
        return 0

    lax.fori_loop(0, nrows, body, 0)


def _tile_rows(s):
    start = s * RQ
    nch = jnp.where(s == NS - 1, (N - (NS - 1) * RQ) // RC, RQ // RC)
    return start, nch


@functools.cache
def _sc_degree():
    return pl.kernel(
        _sc_degree_body,
        out_type=jax.ShapeDtypeStruct((NC * N, DEGW), jnp.float32),
        mesh=_mesh(),
        scratch_types=[
            [pltpu.VMEM((CH,), jnp.int32) for _ in range(3)],
            pltpu.VMEM((CH, DEGW), jnp.float32),
            pltpu.VMEM_SHARED((N, DEGW), jnp.float32),
            [pltpu.SemaphoreType.DMA for _ in range(3)],
            [pltpu.SemaphoreType.DMA for _ in range(3)],
        ],
    )


def _sc_degree_body(dst_hbm, zeros_hbm, out_hbm, didx, ones, degsh,
                    semd, sems):
    c = lax.axis_index("c")
    s = lax.axis_index("s")
    base = (c * NS + s) * EPT

    one16 = jnp.ones((16,), jnp.float32)

    def fill(i, _):
        for j in range(DEGW // 16):
            ones[i, pl.ds(16 * j, 16)] = one16
        return 0

    lax.fori_loop(0, CH, fill, 0)
    start, _ = _tile_rows(s)

    @pl.when(s < NS - 1)
    def _():
        pltpu.sync_copy(zeros_hbm.at[pl.ds(start, RQ)], degsh.at[pl.ds(start, RQ)])

    @pl.when(s == NS - 1)
    def _():
        pltpu.sync_copy(zeros_hbm.at[pl.ds(start, RL)], degsh.at[pl.ds(start, RL)])

    plsc.subcore_barrier()

    def fire(j, b):
        pltpu.async_copy(dst_hbm.at[pl.ds(base + j * CH, CH)], didx[b], semd[b])

    def dwait(b):
        pltpu.make_async_copy(dst_hbm.at[pl.ds(0, CH)], didx[b], semd[b]).wait()

    def scat(b):
        pltpu.async_copy(ones, degsh.at[didx[b]], sems[b], add=True)

    def swait(b):
        pltpu.make_async_copy(ones, degsh.at[didx[b]], sems[b]).wait()

    fire(0, 0)
    fire(1, 1)

    def triple(k, _):
        for t in range(3):
            j = 3 * k + t
            b = t
            bp = (t + 2) % 3
            dwait(b)
            if t == 0:
                @pl.when(k > 0)
                def _():
                    swait(bp)
            else:
                swait(bp)
            fire(j + 2, bp)
            scat(b)
        return 0

    lax.fori_loop(0, (NCHUNK - 2) // 3, triple, 0)
    dwait(0)
    swait(2)
    scat(0)
    dwait(1)
    swait(0)
    scat(1)
    swait(1)
    plsc.subcore_barrier()

    @pl.when(s < NS - 1)
    def _():
        pltpu.sync_copy(degsh.at[pl.ds(start, RQ)],
                        out_hbm.at[pl.ds(c * N + start, RQ)])

    @pl.when(s == NS - 1)
    def _():
        pltpu.sync_copy(degsh.at[pl.ds(start, RL)],
                        out_hbm.at[pl.ds(c * N + start, RL)])


@functools.cache
def _sc_agg():
    return pl.kernel(
        _sc_agg_body,
        out_type=jax.ShapeDtypeStruct((NC * N, F), jnp.float32),
        mesh=_mesh(),
        scratch_types=[
            pltpu.VMEM((NCHUNK, CH), jnp.int32),
            [pltpu.VMEM((CH,), jnp.int32) for _ in range(3)],
            [pltpu.VMEM((CH, F), jnp.float32) for _ in range(3)],
            pltpu.VMEM_SHARED((N, F), jnp.float32),
            [pltpu.SemaphoreType.DMA for _ in range(3)],
            [pltpu.SemaphoreType.DMA for _ in range(3)],
            [pltpu.SemaphoreType.DMA for _ in range(3)],
        ],
    )


def _sc_agg_body(u_hbm, src_hbm, dst_hbm, zeros_hbm, out_hbm, sidx, didx,
                 rows, aggsh, semg, semd, sems):
    c = lax.axis_index("c")
    s = lax.axis_index("s")
    wid = c * NS + s
    base = wid * EPT

    start, _ = _tile_rows(s)

    @pl.when(s < NS - 1)
    def _():
        pltpu.sync_copy(zeros_hbm.at[pl.ds(start, RQ)], aggsh.at[pl.ds(start, RQ)])

    @pl.when(s == NS - 1)
    def _():
        pltpu.sync_copy(zeros_hbm.at[pl.ds(start, RL)], aggsh.at[pl.ds(start, RL)])

    pltpu.sync_copy(src_hbm.at[wid], sidx)
    plsc.subcore_barrier()

    def fire(j, b):
        pltpu.async_copy(u_hbm.at[sidx.at[j]], rows[b], semg[b])
        pltpu.async_copy(dst_hbm.at[pl.ds(base + j * CH, CH)], didx[b], semd[b])

    def gwait(b):
        pltpu.make_async_copy(u_hbm.at[pl.ds(0, CH)], rows[b], semg[b]).wait()
        pltpu.make_async_copy(dst_hbm.at[pl.ds(0, CH)], didx[b], semd[b]).wait()

    def scat(b):
        pltpu.async_copy(rows[b], aggsh.at[didx[b]], sems[b], add=True)

    def swait(b):
        pltpu.make_async_copy(rows[b], aggsh.at[didx[b]], sems[b]).wait()

    fire(0, 0)
    fire(1, 1)

    def triple(k, _):
        for t in range(3):
            j = 3 * k + t
            b = t
            bp = (t + 2) % 3
            gwait(b)
            if t == 0:
                @pl.when(k > 0)
                def _():
                    swait(bp)
            else:
                swait(bp)
            fire(j + 2, bp)
            scat(b)
        return 0

    lax.fori_loop(0, (NCHUNK - 2) // 3, triple, 0)
    gwait(0)
    swait(2)
    scat(0)
    gwait(1)
    swait(0)
    scat(1)
    swait(1)
    plsc.subcore_barrier()

    @pl.when(s < NS - 1)
    def _():
        pltpu.sync_copy(aggsh.at[pl.ds(start, RQ)],
                        out_hbm.at[pl.ds(c * N + start, RQ)])

    @pl.when(s == NS - 1)
    def _():
        pltpu.sync_copy(aggsh.at[pl.ds(start, RL)],
                        out_hbm.at[pl.ds(c * N + start, RL)])


R = 400
NB = N // R

_full = lambda shape: pl.BlockSpec(shape, lambda i: (0, 0))
_rows = lambda w: pl.BlockSpec((R, w), lambda i: (i, 0))
_rows0 = pl.BlockSpec((R, F), lambda i: (i, 0))
_rows1 = pl.BlockSpec((R, F), lambda i: (i + NB, 0))


def _k0a_body(x_ref, d0_ref, d1_ref, wet_ref, be_ref, wct_ref,
              y_ref, u_ref, dinv_ref):
    y = jnp.dot(x_ref[...], wet_ref[...], preferred_element_type=jnp.float32)
    y = y + be_ref[...]
    deg = d0_ref[:, 0:1] + d1_ref[:, 0:1] + 1.0
    dinv = lax.rsqrt(deg)
    yw = jnp.dot(y, wct_ref[...], preferred_element_type=jnp.float32)
    y_ref[...] = y
    u_ref[...] = dinv * yw
    dinv_ref[...] = dinv


_k0a = pl.pallas_call(
    _k0a_body,
    grid=(NB,),
    in_specs=[
        _rows0,
        pl.BlockSpec((R, DEGW), lambda i: (i, 0)),
        pl.BlockSpec((R, DEGW), lambda i: (i + NB, 0)),
        _full((F, F)), _full((1, F)), _full((F, F)),
    ],
    out_specs=[_rows(F), _rows(F), _rows(1)],
    out_shape=[
        jax.ShapeDtypeStruct((N, F), jnp.float32),
        jax.ShapeDtypeStruct((N, F), jnp.float32),
        jax.ShapeDtypeStruct((N, 1), jnp.float32),
    ],
)


def _kc_body(y_ref, dinv_ref, wct_ref, wrt_ref, brc_ref, c_ref):
    y = y_ref[...]
    dinv = dinv_ref[...]
    yw = jnp.dot(y, wct_ref[...], preferred_element_type=jnp.float32)
    yr = jnp.dot(y, wrt_ref[...], preferred_element_type=jnp.float32)
    c_ref[...] = (dinv * dinv) * yw - yw + yr + brc_ref[...]


_kc = pl.pallas_call(
    _kc_body,
    grid=(NB,),
    in_specs=[_rows0, _rows(1), _full((F, F)), _full((F, F)), _full((1, F))],
    out_specs=_rows(F),
    out_shape=jax.ShapeDtypeStruct((N, F), jnp.float32),
)


def _kmid_body(h0_ref, h1_ref, cprev_ref, dinv_ref, wct_ref, y_ref, u_ref):
    dinv = dinv_ref[...]
    y = jnp.maximum(dinv * (h0_ref[...] + h1_ref[...]) + cprev_ref[...], 0.0)
    yw = jnp.dot(y, wct_ref[...], preferred_element_type=jnp.float32)
    y_ref[...] = y
    u_ref[...] = dinv * yw


_kmid = pl.pallas_call(
    _kmid_body,
    grid=(NB,),
    in_specs=[_rows0, _rows1, _rows(F), _rows(1), _full((F, F))],
    out_specs=[_rows(F), _rows(F)],
    out_shape=[
        jax.ShapeDtypeStruct((N, F), jnp.float32),
        jax.ShapeDtypeStruct((N, F), jnp.float32),
    ],
)


def _k3_body(h0_ref, h1_ref, c_ref, dinv_ref, b_ref, wo_ref, bo_ref, out_ref):
    y = jnp.maximum(dinv_ref[...] * (h0_ref[...] + h1_ref[...]) + c_ref[...], 0.0)
    v = jnp.dot(y, wo_ref[...], preferred_element_type=jnp.float32) + bo_ref[...]
    gi = lax.broadcasted_iota(jnp.int32, (R, G), 1)
    oh = (b_ref[...] == gi).astype(jnp.float32)
    part = jnp.sum(oh * v, axis=0, keepdims=True)

    @pl.when(pl.program_id(0) == 0)
    def _():
        out_ref[...] = jnp.zeros_like(out_ref)

    out_ref[...] += part


_k3 = pl.pallas_call(
    _k3_body,
    grid=(NB,),
    in_specs=[_rows0, _rows1, _rows(F), _rows(1), _rows(1),
              _full((F, 1)), _full((1, 1))],
    out_specs=pl.BlockSpec((1, G), lambda i: (0, 0)),
    out_shape=jax.ShapeDtypeStruct((1, G), jnp.float32),
)


def kernel(x, edge_index, batch, W_emb, b_emb, W_conv, b_conv, W_res, b_res,
           W_out, b_out):
    src = edge_index[0].astype(jnp.int32)
    dst = edge_index[1].astype(jnp.int32)
    src2 = src.reshape(NW, NCHUNK, CH)
    batch2 = batch.astype(jnp.int32).reshape(N, 1)
    x = x.astype(jnp.float32)
    wet = W_emb.T
    wct = W_conv.T
    wrt = W_res.T
    be = b_emb.reshape(1, F)
    brc = (b_res + b_conv).reshape(1, F)
    wo = W_out.T.reshape(F, 1)
    bo = b_out.reshape(1, 1)

    zrows = jnp.zeros((N, F), jnp.float32)
    degh = _sc_degree()(dst, zrows)
    y, u, dinv = _k0a(x, degh, degh, wet, be, wct)
    for it in range(ITERS):
        h = _sc_agg()(u, src2, dst, zrows)
        c = _kc(y, dinv, wct, wrt, brc)
        if it < ITERS - 1:
            y, u = _kmid(h, h, c, dinv, wct)
        else:
            out = _k3(h, h, c, dinv, batch2, wo, bo)
    return out.reshape(G)


# speedup vs baseline: 1.3716x; 1.0108x over previous
import functools

import jax
import jax.numpy as jnp
from jax import lax
from jax.experimental import pallas as pl
from jax.experimental.pallas import tpu as pltpu
from jax.experimental.pallas import tpu_sc as plsc

N = 10000
E = 320000
F = 128
G = 64
ITERS = 3

NC, NS = 2, 16
NW = NC * NS
EPT = E // NW
CH = 80
NCHUNK = EPT // CH
DEGW = 128

RQ = 624
RC = 16
RL = N - (NS - 1) * RQ

@functools.cache
def _mesh():
    return plsc.VectorSubcoreMesh(core_axis_name="c", subcore_axis_name="s",
                                  num_cores=NC, num_subcores=NS)


def _zero_fill(ref, nrows, width):
    z16 = jnp.zeros((16,), jnp.float32)

    def body(i, _):
        for j in range(width // 16):
            ref[i, pl.ds(16 * j, 16)] = z16
        return 0

    lax.fori_loop(0, nrows, body, 0)


def _tile_rows(s):
    start = s * RQ
    nch = jnp.where(s == NS - 1, (N - (NS - 1) * RQ) // RC, RQ // RC)
    return start, nch


@functools.cache
def _sc_degree():
    return pl.kernel(
        _sc_degree_body,
        out_type=jax.ShapeDtypeStruct((NC * N, DEGW), jnp.float32),
        mesh=_mesh(),
        scratch_types=[
            [pltpu.VMEM((CH,), jnp.int32) for _ in range(3)],
            pltpu.VMEM((CH, DEGW), jnp.float32),
            pltpu.VMEM_SHARED((N, DEGW), jnp.float32),
            [pltpu.SemaphoreType.DMA for _ in range(3)],
            [pltpu.SemaphoreType.DMA for _ in range(3)],
        ],
    )


def _sc_degree_body(dst_hbm, zeros_hbm, out_hbm, didx, ones, degsh,
                    semd, sems):
    c = lax.axis_index("c")
    s = lax.axis_index("s")
    base = (c * NS + s) * EPT

    one16 = jnp.ones((16,), jnp.float32)

    def fill(i, _):
        for j in range(DEGW // 16):
            ones[i, pl.ds(16 * j, 16)] = one16
        return 0

    lax.fori_loop(0, CH, fill, 0)
    start, _ = _tile_rows(s)

    @pl.when(s < NS - 1)
    def _():
        pltpu.sync_copy(zeros_hbm.at[pl.ds(start, RQ)], degsh.at[pl.ds(start, RQ)])

    @pl.when(s == NS - 1)
    def _():
        pltpu.sync_copy(zeros_hbm.at[pl.ds(start, RL)], degsh.at[pl.ds(start, RL)])

    plsc.subcore_barrier()

    def fire(j, b):
        pltpu.async_copy(dst_hbm.at[pl.ds(base + j * CH, CH)], didx[b], semd[b])

    def dwait(b):
        pltpu.make_async_copy(dst_hbm.at[pl.ds(0, CH)], didx[b], semd[b]).wait()

    def scat(b):
        pltpu.async_copy(ones, degsh.at[didx[b]], sems[b], add=True)

    def swait(b):
        pltpu.make_async_copy(ones, degsh.at[didx[b]], sems[b]).wait()

    fire(0, 0)
    fire(1, 1)

    def triple(k, _):
        for t in range(3):
            j = 3 * k + t
            b = t
            bp = (t + 2) % 3
            dwait(b)
            if t == 0:
                @pl.when(k > 0)
                def _():
                    swait(bp)
            else:
                swait(bp)
            fire(j + 2, bp)
            scat(b)
        return 0

    lax.fori_loop(0, (NCHUNK - 2) // 3, triple, 0)
    dwait(0)
    swait(2)
    scat(0)
    dwait(1)
    swait(0)
    scat(1)
    swait(1)
    plsc.subcore_barrier()

    @pl.when(s < NS - 1)
    def _():
        pltpu.sync_copy(degsh.at[pl.ds(start, RQ)],
                        out_hbm.at[pl.ds(c * N + start, RQ)])

    @pl.when(s == NS - 1)
    def _():
        pltpu.sync_copy(degsh.at[pl.ds(start, RL)],
                        out_hbm.at[pl.ds(c * N + start, RL)])


@functools.cache
def _sc_agg():
    return pl.kernel(
        _sc_agg_body,
        out_type=jax.ShapeDtypeStruct((NC * N, F), jnp.float32),
        mesh=_mesh(),
        scratch_types=[
            pltpu.VMEM((NCHUNK, CH), jnp.int32),
            [pltpu.VMEM((CH,), jnp.int32) for _ in range(3)],
            [pltpu.VMEM((CH, F), jnp.float32) for _ in range(3)],
            pltpu.VMEM_SHARED((N, F), jnp.float32),
            [pltpu.SemaphoreType.DMA for _ in range(3)],
            [pltpu.SemaphoreType.DMA for _ in range(3)],
            [pltpu.SemaphoreType.DMA for _ in range(3)],
        ],
    )


def _sc_agg_body(u_hbm, src_hbm, dst_hbm, zeros_hbm, out_hbm, sidx, didx,
                 rows, aggsh, semg, semd, sems):
    c = lax.axis_index("c")
    s = lax.axis_index("s")
    wid = c * NS + s
    base = wid * EPT

    start, _ = _tile_rows(s)

    @pl.when(s < NS - 1)
    def _():
        pltpu.sync_copy(zeros_hbm.at[pl.ds(start, RQ)], aggsh.at[pl.ds(start, RQ)])

    @pl.when(s == NS - 1)
    def _():
        pltpu.sync_copy(zeros_hbm.at[pl.ds(start, RL)], aggsh.at[pl.ds(start, RL)])

    pltpu.sync_copy(src_hbm.at[wid], sidx)
    plsc.subcore_barrier()

    def fire(j, b):
        pltpu.async_copy(u_hbm.at[sidx.at[j]], rows[b], semg[b])
        pltpu.async_copy(dst_hbm.at[pl.ds(base + j * CH, CH)], didx[b], semd[b])

    def gwait(b):
        pltpu.make_async_copy(u_hbm.at[pl.ds(0, CH)], rows[b], semg[b]).wait()
        pltpu.make_async_copy(dst_hbm.at[pl.ds(0, CH)], didx[b], semd[b]).wait()

    def scat(b):
        pltpu.async_copy(rows[b], aggsh.at[didx[b]], sems[b], add=True)

    def swait(b):
        pltpu.make_async_copy(rows[b], aggsh.at[didx[b]], sems[b]).wait()

    fire(0, 0)
    fire(1, 1)

    def triple(k, _):
        for t in range(3):
            j = 3 * k + t
            b = t
            bp = (t + 2) % 3
            gwait(b)
            if t == 0:
                @pl.when(k > 0)
                def _():
                    swait(bp)
            else:
                swait(bp)
            fire(j + 2, bp)
            scat(b)
        return 0

    lax.fori_loop(0, (NCHUNK - 2) // 3, triple, 0)
    gwait(0)
    swait(2)
    scat(0)
    gwait(1)
    swait(0)
    scat(1)
    swait(1)
    plsc.subcore_barrier()

    @pl.when(s < NS - 1)
    def _():
        pltpu.sync_copy(aggsh.at[pl.ds(start, RQ)],
                        out_hbm.at[pl.ds(c * N + start, RQ)])

    @pl.when(s == NS - 1)
    def _():
        pltpu.sync_copy(aggsh.at[pl.ds(start, RL)],
                        out_hbm.at[pl.ds(c * N + start, RL)])


R = 400
NB = N // R

_full = lambda shape: pl.BlockSpec(shape, lambda i: (0, 0))
_rows = lambda w: pl.BlockSpec((R, w), lambda i: (i, 0))
_rows0 = pl.BlockSpec((R, F), lambda i: (i, 0))
_rows1 = pl.BlockSpec((R, F), lambda i: (i + NB, 0))


def _kemb_body(x_ref, wet_ref, be_ref, wct_ref, y_ref, yw_ref):
    y = jnp.dot(x_ref[...], wet_ref[...], preferred_element_type=jnp.float32)
    y = y + be_ref[...]
    y_ref[...] = y
    yw_ref[...] = jnp.dot(y, wct_ref[...], preferred_element_type=jnp.float32)


_kemb = pl.pallas_call(
    _kemb_body,
    grid=(NB,),
    in_specs=[_rows0, _full((F, F)), _full((1, F)), _full((F, F))],
    out_specs=[_rows(F), _rows(F)],
    out_shape=[
        jax.ShapeDtypeStruct((N, F), jnp.float32),
        jax.ShapeDtypeStruct((N, F), jnp.float32),
    ],
)


def _k0b_body(yw_ref, d0_ref, d1_ref, u_ref, dinv_ref):
    deg = d0_ref[:, 0:1] + d1_ref[:, 0:1] + 1.0
    dinv = lax.rsqrt(deg)
    u_ref[...] = dinv * yw_ref[...]
    dinv_ref[...] = dinv


_k0b = pl.pallas_call(
    _k0b_body,
    grid=(NB,),
    in_specs=[
        _rows0,
        pl.BlockSpec((R, DEGW), lambda i: (i, 0)),
        pl.BlockSpec((R, DEGW), lambda i: (i + NB, 0)),
    ],
    out_specs=[_rows(F), _rows(1)],
    out_shape=[
        jax.ShapeDtypeStruct((N, F), jnp.float32),
        jax.ShapeDtypeStruct((N, 1), jnp.float32),
    ],
)


def _kc_body(y_ref, dinv_ref, wct_ref, wrt_ref, brc_ref, c_ref):
    y = y_ref[...]
    dinv = dinv_ref[...]
    yw = jnp.dot(y, wct_ref[...], preferred_element_type=jnp.float32)
    yr = jnp.dot(y, wrt_ref[...], preferred_element_type=jnp.float32)
    c_ref[...] = (dinv * dinv) * yw - yw + yr + brc_ref[...]


_kc = pl.pallas_call(
    _kc_body,
    grid=(NB,),
    in_specs=[_rows0, _rows(1), _full((F, F)), _full((F, F)), _full((1, F))],
    out_specs=_rows(F),
    out_shape=jax.ShapeDtypeStruct((N, F), jnp.float32),
)


def _kmid_body(h0_ref, h1_ref, cprev_ref, dinv_ref, wct_ref, y_ref, u_ref):
    dinv = dinv_ref[...]
    y = jnp.maximum(dinv * (h0_ref[...] + h1_ref[...]) + cprev_ref[...], 0.0)
    yw = jnp.dot(y, wct_ref[...], preferred_element_type=jnp.float32)
    y_ref[...] = y
    u_ref[...] = dinv * yw


_kmid = pl.pallas_call(
    _kmid_body,
    grid=(NB,),
    in_specs=[_rows0, _rows1, _rows(F), _rows(1), _full((F, F))],
    out_specs=[_rows(F), _rows(F)],
    out_shape=[
        jax.ShapeDtypeStruct((N, F), jnp.float32),
        jax.ShapeDtypeStruct((N, F), jnp.float32),
    ],
)


def _k3_body(h0_ref, h1_ref, c_ref, dinv_ref, b_ref, wo_ref, bo_ref, out_ref):
    y = jnp.maximum(dinv_ref[...] * (h0_ref[...] + h1_ref[...]) + c_ref[...], 0.0)
    v = jnp.dot(y, wo_ref[...], preferred_element_type=jnp.float32) + bo_ref[...]
    gi = lax.broadcasted_iota(jnp.int32, (R, G), 1)
    oh = (b_ref[...] == gi).astype(jnp.float32)
    part = jnp.sum(oh * v, axis=0, keepdims=True)

    @pl.when(pl.program_id(0) == 0)
    def _():
        out_ref[...] = jnp.zeros_like(out_ref)

    out_ref[...] += part


_k3 = pl.pallas_call(
    _k3_body,
    grid=(NB,),
    in_specs=[_rows0, _rows1, _rows(F), _rows(1), _rows(1),
              _full((F, 1)), _full((1, 1))],
    out_specs=pl.BlockSpec((1, G), lambda i: (0, 0)),
    out_shape=jax.ShapeDtypeStruct((1, G), jnp.float32),
)


def kernel(x, edge_index, batch, W_emb, b_emb, W_conv, b_conv, W_res, b_res,
           W_out, b_out):
    src = edge_index[0].astype(jnp.int32)
    dst = edge_index[1].astype(jnp.int32)
    src2 = src.reshape(NW, NCHUNK, CH)
    batch2 = batch.astype(jnp.int32).reshape(N, 1)
    x = x.astype(jnp.float32)
    wet = W_emb.T
    wct = W_conv.T
    wrt = W_res.T
    be = b_emb.reshape(1, F)
    brc = (b_res + b_conv).reshape(1, F)
    wo = W_out.T.reshape(F, 1)
    bo = b_out.reshape(1, 1)

    zrows = jnp.zeros((N, F), jnp.float32)
    degh = _sc_degree()(dst, zrows)
    y, yw0 = _kemb(x, wet, be, wct)
    u, dinv = _k0b(yw0, degh, degh)
    for it in range(ITERS):
        h = _sc_agg()(u, src2, dst, zrows)
        c = _kc(y, dinv, wct, wrt, brc)
        if it < ITERS - 1:
            y, u = _kmid(h, h, c, dinv, wct)
        else:
            out = _k3(h, h, c, dinv, batch2, wo, bo)
    return out.reshape(G)


# speedup vs baseline: 1.3717x; 1.0001x over previous
import functools

import jax
import jax.numpy as jnp
from jax import lax
from jax.experimental import pallas as pl
from jax.experimental.pallas import tpu as pltpu
from jax.experimental.pallas import tpu_sc as plsc

N = 10000
E = 320000
F = 128
G = 64
ITERS = 3

NC, NS = 2, 16
NW = NC * NS
EPT = E // NW
CH = 80
NCHUNK = EPT // CH
DEGW = 128

RQ = 624
RL = N - (NS - 1) * RQ

@functools.cache
def _mesh():
    return plsc.VectorSubcoreMesh(core_axis_name="c", subcore_axis_name="s",
                                  num_cores=NC, num_subcores=NS)


@functools.cache
def _sc_degree():
    return pl.kernel(
        _sc_degree_body,
        out_type=jax.ShapeDtypeStruct((NC * N, DEGW), jnp.float32),
        mesh=_mesh(),
        scratch_types=[
            [pltpu.VMEM((CH,), jnp.int32) for _ in range(3)],
            pltpu.VMEM((CH, DEGW), jnp.float32),
            pltpu.VMEM_SHARED((N, DEGW), jnp.float32),
            [pltpu.SemaphoreType.DMA for _ in range(3)],
            [pltpu.SemaphoreType.DMA for _ in range(3)],
        ],
    )


def _sc_degree_body(dst_hbm, zeros_hbm, out_hbm, didx, ones, degsh,
                    semd, sems):
    c = lax.axis_index("c")
    s = lax.axis_index("s")
    base = (c * NS + s) * EPT

    one16 = jnp.ones((16,), jnp.float32)

    def fill(i, _):
        for j in range(DEGW // 16):
            ones[i, pl.ds(16 * j, 16)] = one16
        return 0

    lax.fori_loop(0, CH, fill, 0)
    start = s * RQ

    @pl.when(s < NS - 1)
    def _():
        pltpu.sync_copy(zeros_hbm.at[pl.ds(start, RQ)], degsh.at[pl.ds(start, RQ)])

    @pl.when(s == NS - 1)
    def _():
        pltpu.sync_copy(zeros_hbm.at[pl.ds(start, RL)], degsh.at[pl.ds(start, RL)])

    plsc.subcore_barrier()

    def fire(j, b):
        pltpu.async_copy(dst_hbm.at[pl.ds(base + j * CH, CH)], didx[b], semd[b])

    def dwait(b):
        pltpu.make_async_copy(dst_hbm.at[pl.ds(0, CH)], didx[b], semd[b]).wait()

    def scat(b):
        pltpu.async_copy(ones, degsh.at[didx[b]], sems[b], add=True)

    def swait(b):
        pltpu.make_async_copy(ones, degsh.at[didx[b]], sems[b]).wait()

    fire(0, 0)
    fire(1, 1)

    def triple(k, _):
        for t in range(3):
            j = 3 * k + t
            b = t
            bp = (t + 2) % 3
            dwait(b)
            if t == 0:
                @pl.when(k > 0)
                def _():
                    swait(bp)
            else:
                swait(bp)
            fire(j + 2, bp)
            scat(b)
        return 0

    lax.fori_loop(0, (NCHUNK - 2) // 3, triple, 0)
    dwait(0)
    swait(2)
    scat(0)
    dwait(1)
    swait(0)
    scat(1)
    swait(1)
    plsc.subcore_barrier()

    @pl.when(s < NS - 1)
    def _():
        pltpu.sync_copy(degsh.at[pl.ds(start, RQ)],
                        out_hbm.at[pl.ds(c * N + start, RQ)])

    @pl.when(s == NS - 1)
    def _():
        pltpu.sync_copy(degsh.at[pl.ds(start, RL)],
                        out_hbm.at[pl.ds(c * N + start, RL)])


@functools.cache
def _sc_agg():
    return pl.kernel(
        _sc_agg_body,
        out_type=jax.ShapeDtypeStruct((NC * N, F), jnp.float32),
        mesh=_mesh(),
        scratch_types=[
            pltpu.VMEM((NCHUNK, CH), jnp.int32),
            [pltpu.VMEM((CH,), jnp.int32) for _ in range(3)],
            [pltpu.VMEM((CH, F), jnp.float32) for _ in range(3)],
            pltpu.VMEM_SHARED((N, F), jnp.float32),
            [pltpu.SemaphoreType.DMA for _ in range(3)],
            [pltpu.SemaphoreType.DMA for _ in range(3)],
            [pltpu.SemaphoreType.DMA for _ in range(3)],
        ],
    )


def _sc_agg_body(u_hbm, src_hbm, dst_hbm, zeros_hbm, out_hbm, sidx, didx,
                 rows, aggsh, semg, semd, sems):
    c = lax.axis_index("c")
    s = lax.axis_index("s")
    wid = c * NS + s
    base = wid * EPT

    start = s * RQ

    @pl.when(s < NS - 1)
    def _():
        pltpu.sync_copy(zeros_hbm.at[pl.ds(start, RQ)], aggsh.at[pl.ds(start, RQ)])

    @pl.when(s == NS - 1)
    def _():
        pltpu.sync_copy(zeros_hbm.at[pl.ds(start, RL)], aggsh.at[pl.ds(start, RL)])

    pltpu.sync_copy(src_hbm.at[wid], sidx)
    plsc.subcore_barrier()

    def fire(j, b):
        pltpu.async_copy(u_hbm.at[sidx.at[j]], rows[b], semg[b])
        pltpu.async_copy(dst_hbm.at[pl.ds(base + j * CH, CH)], didx[b], semd[b])

    def gwait(b):
        pltpu.make_async_copy(u_hbm.at[pl.ds(0, CH)], rows[b], semg[b]).wait()
        pltpu.make_async_copy(dst_hbm.at[pl.ds(0, CH)], didx[b], semd[b]).wait()

    def scat(b):
        pltpu.async_copy(rows[b], aggsh.at[didx[b]], sems[b], add=True)

    def swait(b):
        pltpu.make_async_copy(rows[b], aggsh.at[didx[b]], sems[b]).wait()

    fire(0, 0)
    fire(1, 1)

    def triple(k, _):
        for t in range(3):
            j = 3 * k + t
            b = t
            bp = (t + 2) % 3
            gwait(b)
            if t == 0:
                @pl.when(k > 0)
                def _():
                    swait(bp)
            else:
                swait(bp)
            fire(j + 2, bp)
            scat(b)
        return 0

    lax.fori_loop(0, (NCHUNK - 2) // 3, triple, 0)
    gwait(0)
    swait(2)
    scat(0)
    gwait(1)
    swait(0)
    scat(1)
    swait(1)
    plsc.subcore_barrier()

    @pl.when(s < NS - 1)
    def _():
        pltpu.sync_copy(aggsh.at[pl.ds(start, RQ)],
                        out_hbm.at[pl.ds(c * N + start, RQ)])

    @pl.when(s == NS - 1)
    def _():
        pltpu.sync_copy(aggsh.at[pl.ds(start, RL)],
                        out_hbm.at[pl.ds(c * N + start, RL)])


R = 400
NB = N // R

_full = lambda shape: pl.BlockSpec(shape, lambda i: (0, 0))
_rows = lambda w: pl.BlockSpec((R, w), lambda i: (i, 0))
_rows0 = pl.BlockSpec((R, F), lambda i: (i, 0))
_rows1 = pl.BlockSpec((R, F), lambda i: (i + NB, 0))


def _kemb_body(x_ref, wet_ref, be_ref, wct_ref, y_ref, yw_ref):
    y = jnp.dot(x_ref[...], wet_ref[...], preferred_element_type=jnp.float32)
    y = y + be_ref[...]
    y_ref[...] = y
    yw_ref[...] = jnp.dot(y, wct_ref[...], preferred_element_type=jnp.float32)


_kemb = pl.pallas_call(
    _kemb_body,
    grid=(NB,),
    in_specs=[_rows0, _full((F, F)), _full((1, F)), _full((F, F))],
    out_specs=[_rows(F), _rows(F)],
    out_shape=[
        jax.ShapeDtypeStruct((N, F), jnp.float32),
        jax.ShapeDtypeStruct((N, F), jnp.float32),
    ],
)


def _k0b_body(yw_ref, d0_ref, d1_ref, u_ref, dinv_ref):
    deg = d0_ref[:, 0:1] + d1_ref[:, 0:1] + 1.0
    dinv = lax.rsqrt(deg)
    u_ref[...] = dinv * yw_ref[...]
    dinv_ref[...] = dinv


_k0b = pl.pallas_call(
    _k0b_body,
    grid=(NB,),
    in_specs=[
        _rows0,
        pl.BlockSpec((R, DEGW), lambda i: (i, 0)),
        pl.BlockSpec((R, DEGW), lambda i: (i + NB, 0)),
    ],
    out_specs=[_rows(F), _rows(1)],
    out_shape=[
        jax.ShapeDtypeStruct((N, F), jnp.float32),
        jax.ShapeDtypeStruct((N, 1), jnp.float32),
    ],
)


def _kc_body(y_ref, dinv_ref, wct_ref, wrt_ref, brc_ref, c_ref):
    y = y_ref[...]
    dinv = dinv_ref[...]
    yw = jnp.dot(y, wct_ref[...], preferred_element_type=jnp.float32)
    yr = jnp.dot(y, wrt_ref[...], preferred_element_type=jnp.float32)
    c_ref[...] = (dinv * dinv) * yw - yw + yr + brc_ref[...]


_kc = pl.pallas_call(
    _kc_body,
    grid=(NB,),
    in_specs=[_rows0, _rows(1), _full((F, F)), _full((F, F)), _full((1, F))],
    out_specs=_rows(F),
    out_shape=jax.ShapeDtypeStruct((N, F), jnp.float32),
)


def _kmid_body(h0_ref, h1_ref, cprev_ref, dinv_ref, wct_ref, y_ref, u_ref):
    dinv = dinv_ref[...]
    y = jnp.maximum(dinv * (h0_ref[...] + h1_ref[...]) + cprev_ref[...], 0.0)
    yw = jnp.dot(y, wct_ref[...], preferred_element_type=jnp.float32)
    y_ref[...] = y
    u_ref[...] = dinv * yw


_kmid = pl.pallas_call(
    _kmid_body,
    grid=(NB,),
    in_specs=[_rows0, _rows1, _rows(F), _rows(1), _full((F, F))],
    out_specs=[_rows(F), _rows(F)],
    out_shape=[
        jax.ShapeDtypeStruct((N, F), jnp.float32),
        jax.ShapeDtypeStruct((N, F), jnp.float32),
    ],
)


def _k3_body(h0_ref, h1_ref, c_ref, dinv_ref, b_ref, wo_ref, bo_ref, out_ref):
    y = jnp.maximum(dinv_ref[...] * (h0_ref[...] + h1_ref[...]) + c_ref[...], 0.0)
    v = jnp.dot(y, wo_ref[...], preferred_element_type=jnp.float32) + bo_ref[...]
    gi = lax.broadcasted_iota(jnp.int32, (R, G), 1)
    oh = (b_ref[...] == gi).astype(jnp.float32)
    part = jnp.sum(oh * v, axis=0, keepdims=True)

    @pl.when(pl.program_id(0) == 0)
    def _():
        out_ref[...] = jnp.zeros_like(out_ref)

    out_ref[...] += part


_k3 = pl.pallas_call(
    _k3_body,
    grid=(NB,),
    in_specs=[_rows0, _rows1, _rows(F), _rows(1), _rows(1),
              _full((F, 1)), _full((1, 1))],
    out_specs=pl.BlockSpec((1, G), lambda i: (0, 0)),
    out_shape=jax.ShapeDtypeStruct((1, G), jnp.float32),
)


def kernel(x, edge_index, batch, W_emb, b_emb, W_conv, b_conv, W_res, b_res,
           W_out, b_out):
    src = edge_index[0].astype(jnp.int32)
    dst = edge_index[1].astype(jnp.int32)
    src2 = src.reshape(NW, NCHUNK, CH)
    batch2 = batch.astype(jnp.int32).reshape(N, 1)
    x = x.astype(jnp.float32)
    wet = W_emb.T
    wct = W_conv.T
    wrt = W_res.T
    be = b_emb.reshape(1, F)
    brc = (b_res + b_conv).reshape(1, F)
    wo = W_out.T.reshape(F, 1)
    bo = b_out.reshape(1, 1)

    zrows = jnp.zeros((N, F), jnp.float32)
    degh = _sc_degree()(dst, zrows)
    y, yw0 = _kemb(x, wet, be, wct)
    u, dinv = _k0b(yw0, degh, degh)
    for it in range(ITERS):
        h = _sc_agg()(u, src2, dst, zrows)
        c = _kc(y, dinv, wct, wrt, brc)
        if it < ITERS - 1:
            y, u = _kmid(h, h, c, dinv, wct)
        else:
            out = _k3(h, h, c, dinv, batch2, wo, bo)
    return out.reshape(G)
